# baseline (speedup 1.0000x reference)
import functools

import jax
import jax.numpy as jnp
from jax import lax
from jax.experimental import pallas as pl
from jax.experimental.pallas import tpu as pltpu
from jax.experimental.pallas import tpu_sc as plsc

_B, _G = 4, 96
_A, _KB = 16, 8
_ZB, _YB = 20, 28
_NW = 32
_ITEMS = _B * (_G // _A) * (_G // _KB)
_IPW = _ITEMS // _NW
_NKB = _G // _KB
_ROWW = _G * 2
_BROW = _YB * _ROWW
_BANDW = _ZB * _BROW


def _fold_theta(theta):
    t = theta.reshape(_B, 3, 4).astype(jnp.float32)
    tb = t.astype(jnp.bfloat16).astype(jnp.float32)
    coefs = jnp.concatenate(
        [tb.reshape(_B, 12), jnp.zeros((_B, 4), jnp.float32)], axis=1)
    lat = jnp.linspace(-1.0, 1.0, _G).astype(jnp.float32)
    lat = lat.astype(jnp.bfloat16).astype(jnp.float32)
    lat = jnp.concatenate([lat, jnp.zeros((16,), jnp.float32)])

    sc = jnp.float32(_G) / jnp.float32(_G - 1)
    al = sc * t[:, :, 0]
    ga = sc * t[:, :, 1]
    be = sc * t[:, :, 2]
    de = 0.5 * _G * (t[:, :, 3] - t[:, :, 0] - t[:, :, 1] - t[:, :, 2]) + 0.5 * _G

    g = jnp.arange(_ITEMS, dtype=jnp.int32)
    nb_a, nb_k = _G // _A, _G // _KB
    b = g // (nb_a * nb_k)
    a0 = ((g // nb_k) % nb_a) * _A
    k0 = (g % nb_k) * _KB
    a0f, k0f = a0.astype(jnp.float32), k0.astype(jnp.float32)

    def lo(u, width):
        alu, beu, gau, deu = al[b, u], be[b, u], ga[b, u], de[b, u]
        mn = (deu + jnp.minimum(0.0, (_G - 1.0) * alu)
              + jnp.minimum(beu * k0f, beu * (k0f + _KB - 1))
              + jnp.minimum(gau * a0f, gau * (a0f + _A - 1)))
        return jnp.clip(jnp.floor(mn - 0.8), 0, _G - width).astype(jnp.int32)

    z = jnp.zeros_like(g)
    ikb = g % nb_k
    orow0 = ikb * (_B * _G) + b * _G + a0
    params = jnp.stack(
        [b, a0, k0, lo(2, _ZB), lo(1, _YB), orow0,
         z, z, z, z, z, z, z, z, z, z], axis=-1)
    return coefs, params.reshape(_NW, _IPW, 16), lat


_mesh = plsc.VectorSubcoreMesh(core_axis_name="c", subcore_axis_name="s")

_SCRATCH = [
    pltpu.VMEM((_IPW, 16), jnp.int32),
    pltpu.VMEM((16,), jnp.float32),
    pltpu.VMEM((18, 16), jnp.float32),
    pltpu.VMEM((112,), jnp.float32),
    pltpu.VMEM((_BANDW,), jnp.float32),
    pltpu.VMEM((_G * _KB * 2,), jnp.float32),
    pltpu.SemaphoreType.DMA,
]

_OROW = _G * _KB * 2


def _stn_body(x1, coefs, params, lath, out,
              params_v, coef_v, jb, lat_v, band, ob, sem):
    wid = lax.axis_index("s") * 2 + lax.axis_index("c")
    pltpu.sync_copy(params.at[wid], params_v)
    pltpu.sync_copy(lath, lat_v)
    iotai = lax.iota(jnp.int32, 16)
    c1v = jnp.full((16,), 1, jnp.int32)
    j16c = [iotai * 16 + jnp.int32(256 * jv) for jv in range(6)]

    def item_body(it, carry):
        prow = params_v[it]
        b = prow[0]
        a0 = prow[1]
        k0 = prow[2]
        zb0 = prow[3]
        yb0 = prow[4]
        srow = prow[5]
        pltpu.sync_copy(coefs.at[b], coef_v)
        off0 = (b * _G + zb0) * (_G * _ROWW) + yb0 * _ROWW
        handles = [
            pltpu.async_copy(x1.at[pl.ds(off0 + iz * (_G * _ROWW), _BROW)],
                             band.at[pl.ds(iz * _BROW, _BROW)], sem)
            for iz in range(_ZB)
        ]
        cv = coef_v[:]
        tx0, tx1, tx2, tx3 = cv[0], cv[1], cv[2], cv[3]
        ty0, ty1, ty2, ty3 = cv[4], cv[5], cv[6], cv[7]
        tz0, tz1, tz2, tz3 = cv[8], cv[9], cv[10], cv[11]
        for jv in range(6):
            lj = lat_v[pl.ds(16 * jv, 16)]
            jb[3 * jv + 0] = lj * jnp.full((16,), tx0, jnp.float32)
            jb[3 * jv + 1] = lj * jnp.full((16,), ty0, jnp.float32)
            jb[3 * jv + 2] = lj * jnp.full((16,), tz0, jnp.float32)
        s3x = jnp.full((16,), tx3, jnp.float32)
        s3y = jnp.full((16,), ty3, jnp.float32)
        s3z = jnp.full((16,), tz3, jnp.float32)
        zb0v = jnp.full((16,), zb0, jnp.int32)
        yb0v = jnp.full((16,), yb0, jnp.int32)
        for h in handles:
            h.wait()

        def a_body(ia, carry_a):
            la = lat_v[pl.ds(a0 + ia, 16)][0]
            sax = jnp.full((16,), tx1 * la, jnp.float32)
            say = jnp.full((16,), ty1 * la, jnp.float32)
            saz = jnp.full((16,), tz1 * la, jnp.float32)

            def k_body(ik, carry_k):
                lk = lat_v[pl.ds(k0 + ik, 16)][0]
                skx = jnp.full((16,), tx2 * lk, jnp.float32)
                sky = jnp.full((16,), ty2 * lk, jnp.float32)
                skz = jnp.full((16,), tz2 * lk, jnp.float32)
                k2v = jnp.full((16,), ik * 2, jnp.int32)
                for jv in range(6):
                    xs = ((jb[3 * jv + 0] + sax) + skx) + s3x
                    ys = ((jb[3 * jv + 1] + say) + sky) + s3y
                    zs = ((jb[3 * jv + 2] + saz) + skz) + s3z
                    xp = ((xs + 1.0) * 0.5) * jnp.float32(_G)
                    yp = ((ys + 1.0) * 0.5) * jnp.float32(_G)
                    zp = ((zs + 1.0) * 0.5) * jnp.float32(_G)
                    xr = xp.astype(jnp.int32)
                    yr = yp.astype(jnp.int32)
                    zr = zp.astype(jnp.int32)
                    x0i = xr - (xr.astype(jnp.float32) > xp).astype(jnp.int32)
                    y0i = yr - (yr.astype(jnp.float32) > yp).astype(jnp.int32)
                    z0i = zr - (zr.astype(jnp.float32) > zp).astype(jnp.int32)
                    x0c = jnp.minimum(jnp.maximum(x0i, 0), _G - 1)
                    x1c = jnp.minimum(jnp.maximum(x0i + 1, 0), _G - 1)
                    y0c = jnp.minimum(jnp.maximum(y0i, 0), _G - 1)
                    y1c = jnp.minimum(jnp.maximum(y0i + 1, 0), _G - 1)
                    z0c = jnp.minimum(jnp.maximum(z0i, 0), _G - 1)
                    z1c = jnp.minimum(jnp.maximum(z0i + 1, 0), _G - 1)
                    dx0 = xp - x0c.astype(jnp.float32)
                    dx1 = x1c.astype(jnp.float32) - xp
                    dy0 = yp - y0c.astype(jnp.float32)
                    dy1 = y1c.astype(jnp.float32) - yp
                    dzn = (z1c - z0c).astype(jnp.float32)
                    dz1 = z1c.astype(jnp.float32) - zp
                    p11 = dy1 * dz1
                    p01 = dy0 * dz1
                    p10 = dy1 * dzn
                    p00 = dy0 * dzn
                    z0l = (z0c - zb0v) * _BROW
                    z1l = (z1c - zb0v) * _BROW
                    y0l = (y0c - yb0v) * _ROWW
                    y1l = (y1c - yb0v) * _ROWW
                    r00 = z0l + y0l
                    r01 = z0l + y1l
                    r10 = z1l + y0l
                    r11 = z1l + y1l
                    xc0 = x0c + x0c
                    xc1 = x1c + x1c
                    i_a = r00 + xc0
                    i_b = r01 + xc0
                    i_c = r00 + xc1
                    i_d = r01 + xc1
                    i_e = r10 + xc0
                    i_f = r11 + xc0
                    i_g = r10 + xc1
                    i_h = r11 + xc1
                    wa = dx1 * p11
                    wb = dx1 * p01
                    wc = dx0 * p11
                    wd = dx0 * p01
                    we = dx1 * p10
                    wf = dx1 * p00
                    wg = dx0 * p10
                    wh = dx0 * p00
                    acc0 = wa * plsc.load_gather(band, [i_a])
                    acc0 = acc0 + wb * plsc.load_gather(band, [i_b])
                    acc0 = acc0 + wc * plsc.load_gather(band, [i_c])
                    acc0 = acc0 + wd * plsc.load_gather(band, [i_d])
                    acc0 = acc0 + we * plsc.load_gather(band, [i_e])
                    acc0 = acc0 + wf * plsc.load_gather(band, [i_f])
                    acc0 = acc0 + wg * plsc.load_gather(band, [i_g])
                    acc0 = acc0 + wh * plsc.load_gather(band, [i_h])
                    acc1 = wa * plsc.load_gather(band, [i_a + c1v])
                    acc1 = acc1 + wb * plsc.load_gather(band, [i_b + c1v])
                    acc1 = acc1 + wc * plsc.load_gather(band, [i_c + c1v])
                    acc1 = acc1 + wd * plsc.load_gather(band, [i_d + c1v])
                    acc1 = acc1 + we * plsc.load_gather(band, [i_e + c1v])
                    acc1 = acc1 + wf * plsc.load_gather(band, [i_f + c1v])
                    acc1 = acc1 + wg * plsc.load_gather(band, [i_g + c1v])
                    acc1 = acc1 + wh * plsc.load_gather(band, [i_h + c1v])
                    fi0 = j16c[jv] + k2v
                    plsc.store_scatter(ob, [fi0], acc0)
                    plsc.store_scatter(ob, [fi0 + c1v], acc1)
                return carry_k

            lax.fori_loop(0, _KB, k_body, 0)
            pltpu.sync_copy(ob, out.at[pl.ds((srow + ia) * _OROW, _OROW)])
            return carry_a

        lax.fori_loop(0, _A, a_body, 0)
        return carry

    lax.fori_loop(0, _IPW, item_body, 0)


_stn_kernel = pl.kernel(
    _stn_body,
    mesh=_mesh,
    compiler_params=pltpu.CompilerParams(
        use_tc_tiling_on_sc=False, needs_layout_passes=False),
    out_type=jax.ShapeDtypeStruct((_NKB * _B * _G * _OROW,), jnp.float32),
    scratch_types=_SCRATCH,
)


def kernel(x, theta):
    coefs, params, lat = _fold_theta(theta)
    one = jnp.float32(1.0) + 0.0 * theta.astype(jnp.float32).sum()
    x1 = (x.astype(jnp.float32) * one).reshape(-1)
    y = _stn_kernel(x1, coefs, params, lat)
    y = y.reshape(_NKB, _B * _G, _G, _KB, 2).transpose(1, 2, 0, 3, 4) * one
    return y.reshape(_B, _G, _G, _G, 2)


# speedup vs baseline: 1.4446x; 1.4446x over previous
import functools

import jax
import jax.numpy as jnp
from jax import lax
from jax.experimental import pallas as pl
from jax.experimental.pallas import tpu as pltpu
from jax.experimental.pallas import tpu_sc as plsc

_B, _G = 4, 96
_A, _KB = 8, 16
_ZB, _YB = 28, 20
_NW = 32
_ITEMS = _B * (_G // _A) * (_G // _KB)
_IPW = _ITEMS // _NW
_NKB = _G // _KB
_ROWW = _G * 2
_BROW = _YB * _ROWW
_BANDW = _ZB * _BROW


def _fold_theta(theta):
    t = theta.reshape(_B, 3, 4).astype(jnp.float32)
    tb = t.astype(jnp.bfloat16).astype(jnp.float32)
    coefs = jnp.concatenate(
        [tb.reshape(_B, 12), jnp.zeros((_B, 4), jnp.float32)], axis=1)
    lat = jnp.linspace(-1.0, 1.0, _G).astype(jnp.float32)
    lat = lat.astype(jnp.bfloat16).astype(jnp.float32)
    lat = jnp.concatenate([lat, jnp.zeros((16,), jnp.float32)])

    sc = jnp.float32(_G) / jnp.float32(_G - 1)
    al = sc * t[:, :, 0]
    ga = sc * t[:, :, 1]
    be = sc * t[:, :, 2]
    de = 0.5 * _G * (t[:, :, 3] - t[:, :, 0] - t[:, :, 1] - t[:, :, 2]) + 0.5 * _G

    g = jnp.arange(_ITEMS, dtype=jnp.int32)
    nb_a, nb_k = _G // _A, _G // _KB
    b = g // (nb_a * nb_k)
    a0 = ((g // nb_k) % nb_a) * _A
    k0 = (g % nb_k) * _KB
    a0f, k0f = a0.astype(jnp.float32), k0.astype(jnp.float32)

    def lo(u, width):
        alu, beu, gau, deu = al[b, u], be[b, u], ga[b, u], de[b, u]
        mn = (deu + jnp.minimum(0.0, (_G - 1.0) * alu)
              + jnp.minimum(beu * k0f, beu * (k0f + _KB - 1))
              + jnp.minimum(gau * a0f, gau * (a0f + _A - 1)))
        return jnp.clip(jnp.floor(mn - 0.8), 0, _G - width).astype(jnp.int32)

    z = jnp.zeros_like(g)
    params = jnp.stack(
        [b, a0, k0, lo(2, _ZB), lo(1, _YB), b * _G + a0,
         z, z, z, z, z, z, z, z, z, z], axis=-1)
    return coefs, params.reshape(_NW, _IPW, 16), lat


_mesh = plsc.VectorSubcoreMesh(core_axis_name="c", subcore_axis_name="s")

_SCRATCH = [
    pltpu.VMEM((_IPW, 16), jnp.int32),
    pltpu.VMEM((16,), jnp.float32),
    pltpu.VMEM((18, 16), jnp.float32),
    pltpu.VMEM((112,), jnp.float32),
    pltpu.VMEM((_BANDW,), jnp.float32),
    pltpu.VMEM((_G, 2, _KB), jnp.float32),
    pltpu.SemaphoreType.DMA,
]


def _stn_body(x1, coefs, params, lath, out,
              params_v, coef_v, jb, lat_v, band, ob, sem):
    wid = lax.axis_index("s") * 2 + lax.axis_index("c")
    pltpu.sync_copy(params.at[wid], params_v)
    pltpu.sync_copy(lath, lat_v)
    iotai = lax.iota(jnp.int32, 16)
    c0v = jnp.zeros((16,), jnp.int32)
    c1v = jnp.full((16,), 1, jnp.int32)
    jvecs = [iotai + jnp.int32(16 * jv) for jv in range(6)]

    def item_body(it, carry):
        prow = params_v[it]
        b = prow[0]
        a0 = prow[1]
        k0 = pl.multiple_of(prow[2], _KB)
        zb0 = prow[3]
        yb0 = prow[4]
        srow = prow[5]
        pltpu.sync_copy(coefs.at[b], coef_v)
        off0 = (b * _G + zb0) * (_G * _ROWW) + yb0 * _ROWW
        handles = [
            pltpu.async_copy(x1.at[pl.ds(off0 + iz * (_G * _ROWW), _BROW)],
                             band.at[pl.ds(iz * _BROW, _BROW)], sem)
            for iz in range(_ZB)
        ]
        cv = coef_v[:]
        tx0, tx1, tx2, tx3 = cv[0], cv[1], cv[2], cv[3]
        ty0, ty1, ty2, ty3 = cv[4], cv[5], cv[6], cv[7]
        tz0, tz1, tz2, tz3 = cv[8], cv[9], cv[10], cv[11]
        for jv in range(6):
            lj = lat_v[pl.ds(16 * jv, 16)]
            jb[3 * jv + 0] = lj * jnp.full((16,), tx0, jnp.float32)
            jb[3 * jv + 1] = lj * jnp.full((16,), ty0, jnp.float32)
            jb[3 * jv + 2] = lj * jnp.full((16,), tz0, jnp.float32)
        s3x = jnp.full((16,), tx3, jnp.float32)
        s3y = jnp.full((16,), ty3, jnp.float32)
        s3z = jnp.full((16,), tz3, jnp.float32)
        zb0v = jnp.full((16,), zb0, jnp.int32)
        yb0v = jnp.full((16,), yb0, jnp.int32)
        for h in handles:
            h.wait()

        def a_body(ia, carry_a):
            la = lat_v[pl.ds(a0 + ia, 16)][0]
            sax = jnp.full((16,), tx1 * la, jnp.float32)
            say = jnp.full((16,), ty1 * la, jnp.float32)
            saz = jnp.full((16,), tz1 * la, jnp.float32)

            def k_body(ik, carry_k):
                lk = lat_v[pl.ds(k0 + ik, 16)][0]
                skx = jnp.full((16,), tx2 * lk, jnp.float32)
                sky = jnp.full((16,), ty2 * lk, jnp.float32)
                skz = jnp.full((16,), tz2 * lk, jnp.float32)
                kv = jnp.full((16,), ik, jnp.int32)
                for jv in range(6):
                    xs = ((jb[3 * jv + 0] + sax) + skx) + s3x
                    ys = ((jb[3 * jv + 1] + say) + sky) + s3y
                    zs = ((jb[3 * jv + 2] + saz) + skz) + s3z
                    xp = ((xs + 1.0) * 0.5) * jnp.float32(_G)
                    yp = ((ys + 1.0) * 0.5) * jnp.float32(_G)
                    zp = ((zs + 1.0) * 0.5) * jnp.float32(_G)
                    xr = xp.astype(jnp.int32)
                    yr = yp.astype(jnp.int32)
                    zr = zp.astype(jnp.int32)
                    x0i = xr - (xr.astype(jnp.float32) > xp).astype(jnp.int32)
                    y0i = yr - (yr.astype(jnp.float32) > yp).astype(jnp.int32)
                    z0i = zr - (zr.astype(jnp.float32) > zp).astype(jnp.int32)
                    x0c = jnp.minimum(jnp.maximum(x0i, 0), _G - 1)
                    x1c = jnp.minimum(jnp.maximum(x0i + 1, 0), _G - 1)
                    y0c = jnp.minimum(jnp.maximum(y0i, 0), _G - 1)
                    y1c = jnp.minimum(jnp.maximum(y0i + 1, 0), _G - 1)
                    z0c = jnp.minimum(jnp.maximum(z0i, 0), _G - 1)
                    z1c = jnp.minimum(jnp.maximum(z0i + 1, 0), _G - 1)
                    dx0 = xp - x0c.astype(jnp.float32)
                    dx1 = x1c.astype(jnp.float32) - xp
                    dy0 = yp - y0c.astype(jnp.float32)
                    dy1 = y1c.astype(jnp.float32) - yp
                    dzn = (z1c - z0c).astype(jnp.float32)
                    dz1 = z1c.astype(jnp.float32) - zp
                    p11 = dy1 * dz1
                    p01 = dy0 * dz1
                    p10 = dy1 * dzn
                    p00 = dy0 * dzn
                    z0l = (z0c - zb0v) * _BROW
                    z1l = (z1c - zb0v) * _BROW
                    y0l = (y0c - yb0v) * _ROWW
                    y1l = (y1c - yb0v) * _ROWW
                    r00 = z0l + y0l
                    r01 = z0l + y1l
                    r10 = z1l + y0l
                    r11 = z1l + y1l
                    xc0 = x0c + x0c
                    xc1 = x1c + x1c
                    i_a = r00 + xc0
                    i_b = r01 + xc0
                    i_c = r00 + xc1
                    i_d = r01 + xc1
                    i_e = r10 + xc0
                    i_f = r11 + xc0
                    i_g = r10 + xc1
                    i_h = r11 + xc1
                    wa = dx1 * p11
                    wb = dx1 * p01
                    wc = dx0 * p11
                    wd = dx0 * p01
                    we = dx1 * p10
                    wf = dx1 * p00
                    wg = dx0 * p10
                    wh = dx0 * p00
                    acc0 = wa * plsc.load_gather(band, [i_a])
                    acc0 = acc0 + wb * plsc.load_gather(band, [i_b])
                    acc0 = acc0 + wc * plsc.load_gather(band, [i_c])
                    acc0 = acc0 + wd * plsc.load_gather(band, [i_d])
                    acc0 = acc0 + we * plsc.load_gather(band, [i_e])
                    acc0 = acc0 + wf * plsc.load_gather(band, [i_f])
                    acc0 = acc0 + wg * plsc.load_gather(band, [i_g])
                    acc0 = acc0 + wh * plsc.load_gather(band, [i_h])
                    acc1 = wa * plsc.load_gather(band, [i_a + c1v])
                    acc1 = acc1 + wb * plsc.load_gather(band, [i_b + c1v])
                    acc1 = acc1 + wc * plsc.load_gather(band, [i_c + c1v])
                    acc1 = acc1 + wd * plsc.load_gather(band, [i_d + c1v])
                    acc1 = acc1 + we * plsc.load_gather(band, [i_e + c1v])
                    acc1 = acc1 + wf * plsc.load_gather(band, [i_f + c1v])
                    acc1 = acc1 + wg * plsc.load_gather(band, [i_g + c1v])
                    acc1 = acc1 + wh * plsc.load_gather(band, [i_h + c1v])
                    plsc.store_scatter(ob, [jvecs[jv], c0v, kv], acc0)
                    plsc.store_scatter(ob, [jvecs[jv], c1v, kv], acc1)
                return carry_k

            lax.fori_loop(0, _KB, k_body, 0)
            pltpu.sync_copy(
                ob, out.at[pl.ds((srow + ia) * _G, _G), :, pl.ds(k0, _KB)])
            return carry_a

        lax.fori_loop(0, _A, a_body, 0)
        return carry

    lax.fori_loop(0, _IPW, item_body, 0)


_stn_kernel = pl.kernel(
    _stn_body,
    mesh=_mesh,
    compiler_params=pltpu.CompilerParams(
        use_tc_tiling_on_sc=False, needs_layout_passes=False),
    out_type=jax.ShapeDtypeStruct((_B * _G * _G, 2, _G), jnp.float32),
    scratch_types=_SCRATCH,
)


def kernel(x, theta):
    coefs, params, lat = _fold_theta(theta)
    x1 = x.astype(jnp.float32).reshape(-1)
    y = _stn_kernel(x1, coefs, params, lat)
    y = y.reshape(_B, _G, _G, 2, _G).transpose(0, 1, 2, 4, 3)
    return y


# speedup vs baseline: 9.9872x; 6.9137x over previous
import functools

import jax
import jax.numpy as jnp
from jax import lax
from jax.experimental import pallas as pl
from jax.experimental.pallas import tpu as pltpu
from jax.experimental.pallas import tpu_sc as plsc

_B, _G = 4, 96
_A, _KB = 8, 16
_ZB, _YB = 28, 20
_NW = 32
_ITEMS = _B * (_G // _A) * (_G // _KB)
_IPW = _ITEMS // _NW
_NKB = _G // _KB
_ROWW = _G * 2
_BROW = _YB * _ROWW
_BANDW = _ZB * _BROW


def _fold_theta(theta):
    t = theta.reshape(_B, 3, 4).astype(jnp.float32)
    tb = t.astype(jnp.bfloat16).astype(jnp.float32)
    coefs = jnp.concatenate(
        [tb.reshape(_B, 12), jnp.zeros((_B, 4), jnp.float32)], axis=1)
    lat = jnp.linspace(-1.0, 1.0, _G).astype(jnp.float32)
    lat = lat.astype(jnp.bfloat16).astype(jnp.float32)
    lat = jnp.concatenate([lat, jnp.zeros((16,), jnp.float32)])

    sc = jnp.float32(_G) / jnp.float32(_G - 1)
    al = sc * t[:, :, 0]
    ga = sc * t[:, :, 1]
    be = sc * t[:, :, 2]
    de = 0.5 * _G * (t[:, :, 3] - t[:, :, 0] - t[:, :, 1] - t[:, :, 2]) + 0.5 * _G

    g = jnp.arange(_ITEMS, dtype=jnp.int32)
    nb_a, nb_k = _G // _A, _G // _KB
    b = g // (nb_a * nb_k)
    a0 = ((g // nb_k) % nb_a) * _A
    k0 = (g % nb_k) * _KB
    a0f, k0f = a0.astype(jnp.float32), k0.astype(jnp.float32)

    def lo(u, width):
        alu, beu, gau, deu = al[b, u], be[b, u], ga[b, u], de[b, u]
        mn = (deu + jnp.minimum(0.0, (_G - 1.0) * alu)
              + jnp.minimum(beu * k0f, beu * (k0f + _KB - 1))
              + jnp.minimum(gau * a0f, gau * (a0f + _A - 1)))
        return jnp.clip(jnp.floor(mn - 0.8), 0, _G - width).astype(jnp.int32)

    z = jnp.zeros_like(g)
    params = jnp.stack(
        [b, a0, k0, lo(2, _ZB), lo(1, _YB), b * _G + a0,
         z, z, z, z, z, z, z, z, z, z], axis=-1)
    return coefs, params.reshape(_NW, _IPW, 16), lat


_mesh = plsc.VectorSubcoreMesh(core_axis_name="c", subcore_axis_name="s")

_SCRATCH = [
    pltpu.VMEM((_IPW, 16), jnp.int32),
    pltpu.VMEM((16,), jnp.float32),
    pltpu.VMEM((18, 16), jnp.float32),
    pltpu.VMEM((112,), jnp.float32),
    pltpu.VMEM((_ZB * _YB, 2, _G), jnp.float32),
    pltpu.VMEM((_G, 2, _KB), jnp.float32),
    pltpu.SemaphoreType.DMA,
]


def _stn_body(x1, coefs, params, lath, out,
              params_v, coef_v, jb, lat_v, band, ob, sem):
    wid = lax.axis_index("s") * 2 + lax.axis_index("c")
    pltpu.sync_copy(params.at[wid], params_v)
    pltpu.sync_copy(lath, lat_v)
    iotai = lax.iota(jnp.int32, 16)
    c0v = jnp.zeros((16,), jnp.int32)
    c1v = jnp.full((16,), 1, jnp.int32)
    jvecs = [iotai + jnp.int32(16 * jv) for jv in range(6)]

    def item_body(it, carry):
        prow = params_v[it]
        b = prow[0]
        a0 = prow[1]
        k0 = pl.multiple_of(prow[2], _KB)
        zb0 = prow[3]
        yb0 = prow[4]
        srow = prow[5]
        pltpu.sync_copy(coefs.at[b], coef_v)
        handles = [
            pltpu.async_copy(x1.at[b, zb0 + iz, pl.ds(yb0, _YB), :, :],
                             band.at[pl.ds(iz * _YB, _YB), :, :], sem)
            for iz in range(_ZB)
        ]
        cv = coef_v[:]
        tx0, tx1, tx2, tx3 = cv[0], cv[1], cv[2], cv[3]
        ty0, ty1, ty2, ty3 = cv[4], cv[5], cv[6], cv[7]
        tz0, tz1, tz2, tz3 = cv[8], cv[9], cv[10], cv[11]
        for jv in range(6):
            lj = lat_v[pl.ds(16 * jv, 16)]
            jb[3 * jv + 0] = lj * jnp.full((16,), tx0, jnp.float32)
            jb[3 * jv + 1] = lj * jnp.full((16,), ty0, jnp.float32)
            jb[3 * jv + 2] = lj * jnp.full((16,), tz0, jnp.float32)
        s3x = jnp.full((16,), tx3, jnp.float32)
        s3y = jnp.full((16,), ty3, jnp.float32)
        s3z = jnp.full((16,), tz3, jnp.float32)
        zb0v = jnp.full((16,), zb0, jnp.int32)
        yb0v = jnp.full((16,), yb0, jnp.int32)
        for h in handles:
            h.wait()

        def a_body(ia, carry_a):
            la = lat_v[pl.ds(a0 + ia, 16)][0]
            sax = jnp.full((16,), tx1 * la, jnp.float32)
            say = jnp.full((16,), ty1 * la, jnp.float32)
            saz = jnp.full((16,), tz1 * la, jnp.float32)

            def k_body(ik, carry_k):
                lk = lat_v[pl.ds(k0 + ik, 16)][0]
                skx = jnp.full((16,), tx2 * lk, jnp.float32)
                sky = jnp.full((16,), ty2 * lk, jnp.float32)
                skz = jnp.full((16,), tz2 * lk, jnp.float32)
                kv = jnp.full((16,), ik, jnp.int32)
                for jv in range(6):
                    xs = ((jb[3 * jv + 0] + sax) + skx) + s3x
                    ys = ((jb[3 * jv + 1] + say) + sky) + s3y
                    zs = ((jb[3 * jv + 2] + saz) + skz) + s3z
                    xp = ((xs + 1.0) * 0.5) * jnp.float32(_G)
                    yp = ((ys + 1.0) * 0.5) * jnp.float32(_G)
                    zp = ((zs + 1.0) * 0.5) * jnp.float32(_G)
                    xr = xp.astype(jnp.int32)
                    yr = yp.astype(jnp.int32)
                    zr = zp.astype(jnp.int32)
                    x0i = xr - (xr.astype(jnp.float32) > xp).astype(jnp.int32)
                    y0i = yr - (yr.astype(jnp.float32) > yp).astype(jnp.int32)
                    z0i = zr - (zr.astype(jnp.float32) > zp).astype(jnp.int32)
                    x0c = jnp.minimum(jnp.maximum(x0i, 0), _G - 1)
                    x1c = jnp.minimum(jnp.maximum(x0i + 1, 0), _G - 1)
                    y0c = jnp.minimum(jnp.maximum(y0i, 0), _G - 1)
                    y1c = jnp.minimum(jnp.maximum(y0i + 1, 0), _G - 1)
                    z0c = jnp.minimum(jnp.maximum(z0i, 0), _G - 1)
                    z1c = jnp.minimum(jnp.maximum(z0i + 1, 0), _G - 1)
                    dx0 = xp - x0c.astype(jnp.float32)
                    dx1 = x1c.astype(jnp.float32) - xp
                    dy0 = yp - y0c.astype(jnp.float32)
                    dy1 = y1c.astype(jnp.float32) - yp
                    dzn = (z1c - z0c).astype(jnp.float32)
                    dz1 = z1c.astype(jnp.float32) - zp
                    p11 = dy1 * dz1
                    p01 = dy0 * dz1
                    p10 = dy1 * dzn
                    p00 = dy0 * dzn
                    z0l = (z0c - zb0v) * _YB
                    z1l = (z1c - zb0v) * _YB
                    y0l = y0c - yb0v
                    y1l = y1c - yb0v
                    i_a = z0l + y0l
                    i_b = z0l + y1l
                    i_e = z1l + y0l
                    i_f = z1l + y1l
                    wa = dx1 * p11
                    wb = dx1 * p01
                    wc = dx0 * p11
                    wd = dx0 * p01
                    we = dx1 * p10
                    wf = dx1 * p00
                    wg = dx0 * p10
                    wh = dx0 * p00
                    acc0 = wa * plsc.load_gather(band, [i_a, c0v, x0c])
                    acc0 = acc0 + wb * plsc.load_gather(band, [i_b, c0v, x0c])
                    acc0 = acc0 + wc * plsc.load_gather(band, [i_a, c0v, x1c])
                    acc0 = acc0 + wd * plsc.load_gather(band, [i_b, c0v, x1c])
                    acc0 = acc0 + we * plsc.load_gather(band, [i_e, c0v, x0c])
                    acc0 = acc0 + wf * plsc.load_gather(band, [i_f, c0v, x0c])
                    acc0 = acc0 + wg * plsc.load_gather(band, [i_e, c0v, x1c])
                    acc0 = acc0 + wh * plsc.load_gather(band, [i_f, c0v, x1c])
                    acc1 = wa * plsc.load_gather(band, [i_a, c1v, x0c])
                    acc1 = acc1 + wb * plsc.load_gather(band, [i_b, c1v, x0c])
                    acc1 = acc1 + wc * plsc.load_gather(band, [i_a, c1v, x1c])
                    acc1 = acc1 + wd * plsc.load_gather(band, [i_b, c1v, x1c])
                    acc1 = acc1 + we * plsc.load_gather(band, [i_e, c1v, x0c])
                    acc1 = acc1 + wf * plsc.load_gather(band, [i_f, c1v, x0c])
                    acc1 = acc1 + wg * plsc.load_gather(band, [i_e, c1v, x1c])
                    acc1 = acc1 + wh * plsc.load_gather(band, [i_f, c1v, x1c])
                    plsc.store_scatter(ob, [jvecs[jv], c0v, kv], acc0)
                    plsc.store_scatter(ob, [jvecs[jv], c1v, kv], acc1)
                return carry_k

            lax.fori_loop(0, _KB, k_body, 0)
            pltpu.sync_copy(
                ob, out.at[pl.ds((srow + ia) * _G, _G), :, pl.ds(k0, _KB)])
            return carry_a

        lax.fori_loop(0, _A, a_body, 0)
        return carry

    lax.fori_loop(0, _IPW, item_body, 0)


_stn_kernel = pl.kernel(
    _stn_body,
    mesh=_mesh,
    compiler_params=pltpu.CompilerParams(
        use_tc_tiling_on_sc=False, needs_layout_passes=False),
    out_type=jax.ShapeDtypeStruct((_B * _G * _G, 2, _G), jnp.float32),
    scratch_types=_SCRATCH,
)


def kernel(x, theta):
    coefs, params, lat = _fold_theta(theta)
    x1 = x.astype(jnp.float32).transpose(0, 1, 2, 4, 3)
    y = _stn_kernel(x1, coefs, params, lat)
    y = y.reshape(_B, _G, _G, 2, _G).transpose(0, 1, 2, 4, 3)
    return y
